# baseline (speedup 1.0000x reference)
import jax
import jax.numpy as jnp
from jax import lax
from jax.experimental import pallas as pl
from jax.experimental.pallas import tpu as pltpu
from jax.experimental.pallas import tpu_sc as plsc

B = 16384
D = 128
K = D // 2

_NC = 2
_NS = 16
_NW = _NC * _NS
_ROWS_PER_W = B // _NW
_C = 128
_NCHUNK = _ROWS_PER_W // _C
_G = _C // 16
_NBUF = 2
_PITCH = 17


def _body(h_hbm, r_id_hbm, t_hbm, rel_hbm, out_hbm,
          idx_all, out_all, tsc_v,
          h_v0, t_v0, rows_v0, h_v1, t_v1, rows_v1,
          sem_h0, sem_t0, sem_r0, sem_h1, sem_t1, sem_r1):
    wid = lax.axis_index("s") * _NC + lax.axis_index("c")
    lanes = lax.iota(jnp.int32, 16)
    base = wid * _ROWS_PER_W

    bufs = ((h_v0, t_v0, rows_v0, sem_h0, sem_t0, sem_r0),
            (h_v1, t_v1, rows_v1, sem_h1, sem_t1, sem_r1))

    def issue_ht(c):
        h_v, t_v, _, sem_h, sem_t, _ = bufs[c % _NBUF]
        cbase = base + c * _C
        hh = pltpu.async_copy(h_hbm.at[pl.ds(cbase * D, _C * D)], h_v, sem_h)
        ht = pltpu.async_copy(t_hbm.at[pl.ds(cbase * D, _C * D)], t_v, sem_t)
        return hh, ht

    def issue_rel(c):
        _, _, rows_v, _, _, sem_r = bufs[c % _NBUF]
        return pltpu.async_copy(
            rel_hbm.at[idx_all.at[pl.ds(c * _C, _C)]], rows_v, sem_r)

    ht0 = issue_ht(0)
    pltpu.sync_copy(r_id_hbm.at[pl.ds(base, _ROWS_PER_W)], idx_all)
    handles = [ht0 + (issue_rel(0),)]

    for c in range(_NCHUNK):
        h_v, t_v, rows_v = bufs[c % _NBUF][:3]
        if c + 1 < _NCHUNK:
            handles.append(issue_ht(c + 1) + (issue_rel(c + 1),))
        for hd in handles.pop(0):
            hd.wait()

        @plsc.parallel_loop(0, _C, unroll=8)
        def row_body(r):
            row = r * D
            acc1 = None
            acc2 = None
            for j in range(4):
                lo = row + 16 * j
                hi = lo + K
                hr = h_v[pl.ds(lo, 16)]
                him = h_v[pl.ds(hi, 16)]
                tr = t_v[pl.ds(lo, 16)]
                ti = t_v[pl.ds(hi, 16)]
                rr = rows_v[r, pl.ds(16 * j, 16)]
                ri = rows_v[r, pl.ds(K + 16 * j, 16)]
                t1 = rr * (hr * tr + him * ti)
                t2 = ri * (hr * ti - him * tr)
                acc1 = t1 if acc1 is None else acc1 + t1
                acc2 = t2 if acc2 is None else acc2 + t2
            tsc_v[pl.ds(r * _PITCH, 16)] = acc1 + acc2

        @plsc.parallel_loop(0, _G, unroll=2)
        def red_body(g):
            col_base = (g * 16 + lanes) * _PITCH
            parts = [plsc.load_gather(tsc_v, [col_base + j]) for j in range(16)]
            while len(parts) > 1:
                parts = [a + b for a, b in zip(parts[::2], parts[1::2])]
            out_all[pl.ds(c * _C + g * 16, 16)] = parts[0]

    pltpu.sync_copy(out_all, out_hbm.at[pl.ds(base, _ROWS_PER_W)])


@jax.jit
def _complex_score(h, r_id, t, rel_weight):
    mesh = plsc.VectorSubcoreMesh(core_axis_name="c", subcore_axis_name="s")
    return pl.kernel(
        _body,
        mesh=mesh,
        compiler_params=pltpu.CompilerParams(needs_layout_passes=False),
        out_type=jax.ShapeDtypeStruct((B,), jnp.float32),
        scratch_types=[
            pltpu.VMEM((_ROWS_PER_W,), jnp.int32),
            pltpu.VMEM((_ROWS_PER_W,), jnp.float32),
            pltpu.VMEM((_C * _PITCH,), jnp.float32),
            pltpu.VMEM((_C * D,), jnp.float32),
            pltpu.VMEM((_C * D,), jnp.float32),
            pltpu.VMEM((_C, D), jnp.float32),
            pltpu.VMEM((_C * D,), jnp.float32),
            pltpu.VMEM((_C * D,), jnp.float32),
            pltpu.VMEM((_C, D), jnp.float32),
            pltpu.SemaphoreType.DMA,
            pltpu.SemaphoreType.DMA,
            pltpu.SemaphoreType.DMA,
            pltpu.SemaphoreType.DMA,
            pltpu.SemaphoreType.DMA,
            pltpu.SemaphoreType.DMA,
        ],
    )(h.reshape(B * D), r_id, t.reshape(B * D), rel_weight)


def kernel(h, r_id, t, rel_weight):
    return _complex_score(h, r_id, t, rel_weight)


# speedup vs baseline: 1.2738x; 1.2738x over previous
import jax
import jax.numpy as jnp
from jax import lax
from jax.experimental import pallas as pl
from jax.experimental.pallas import tpu as pltpu
from jax.experimental.pallas import tpu_sc as plsc

B = 16384
D = 128
K = D // 2

_NC = 2
_NS = 16
_NW = _NC * _NS
_ROWS_PER_W = B // _NW
_C = 128
_NCHUNK = _ROWS_PER_W // _C
_G = _C // 16
_NBUF = 2
_PITCH = 17


def _body(h_hbm, r_id_hbm, t_hbm, rel_hbm, out_hbm,
          idx_all, out_all, tsc_v,
          h_v0, t_v0, rows_v0, h_v1, t_v1, rows_v1,
          sem_h0, sem_t0, sem_r0, sem_h1, sem_t1, sem_r1):
    wid = lax.axis_index("s") * _NC + lax.axis_index("c")
    lanes = lax.iota(jnp.int32, 16)
    base = wid * _ROWS_PER_W

    bufs = ((h_v0, t_v0, rows_v0, sem_h0, sem_t0, sem_r0),
            (h_v1, t_v1, rows_v1, sem_h1, sem_t1, sem_r1))

    def issue_ht(c):
        h_v, t_v, _, sem_h, sem_t, _ = bufs[c % _NBUF]
        cbase = base + c * _C
        hh = pltpu.async_copy(h_hbm.at[pl.ds(cbase * D, _C * D)], h_v, sem_h)
        ht = pltpu.async_copy(t_hbm.at[pl.ds(cbase * D, _C * D)], t_v, sem_t)
        return hh, ht

    def issue_rel(c):
        _, _, rows_v, _, _, sem_r = bufs[c % _NBUF]
        return pltpu.async_copy(
            rel_hbm.at[idx_all.at[pl.ds(c * _C, _C)]], rows_v, sem_r)

    ht0 = issue_ht(0)
    pltpu.sync_copy(r_id_hbm.at[pl.ds(base, _ROWS_PER_W)], idx_all)
    handles = [ht0 + (issue_rel(0),)]

    for c in range(_NCHUNK):
        h_v, t_v, rows_v = bufs[c % _NBUF][:3]
        if c + 1 < _NCHUNK:
            handles.append(issue_ht(c + 1) + (issue_rel(c + 1),))
        for hd in handles.pop(0):
            hd.wait()

        @plsc.parallel_loop(0, _C, unroll=4)
        def row_body(r):
            row = r * D
            acc1 = None
            acc2 = None
            for j in range(4):
                lo = row + 16 * j
                hi = lo + K
                hr = h_v[pl.ds(lo, 16)]
                him = h_v[pl.ds(hi, 16)]
                tr = t_v[pl.ds(lo, 16)]
                ti = t_v[pl.ds(hi, 16)]
                rr = rows_v[r, pl.ds(16 * j, 16)]
                ri = rows_v[r, pl.ds(K + 16 * j, 16)]
                t1 = rr * (hr * tr + him * ti)
                t2 = ri * (hr * ti - him * tr)
                acc1 = t1 if acc1 is None else acc1 + t1
                acc2 = t2 if acc2 is None else acc2 + t2
            tsc_v[pl.ds(r * _PITCH, 16)] = acc1 + acc2

        @plsc.parallel_loop(0, _G, unroll=2)
        def red_body(g):
            col_base = (g * 16 + lanes) * _PITCH
            parts = [plsc.load_gather(tsc_v, [col_base + j]) for j in range(16)]
            while len(parts) > 1:
                parts = [a + b for a, b in zip(parts[::2], parts[1::2])]
            out_all[pl.ds(c * _C + g * 16, 16)] = parts[0]

    pltpu.sync_copy(out_all, out_hbm.at[pl.ds(base, _ROWS_PER_W)])


@jax.jit
def _complex_score(h, r_id, t, rel_weight):
    mesh = plsc.VectorSubcoreMesh(core_axis_name="c", subcore_axis_name="s")
    return pl.kernel(
        _body,
        mesh=mesh,
        compiler_params=pltpu.CompilerParams(needs_layout_passes=False),
        out_type=jax.ShapeDtypeStruct((B,), jnp.float32),
        scratch_types=[
            pltpu.VMEM((_ROWS_PER_W,), jnp.int32),
            pltpu.VMEM((_ROWS_PER_W,), jnp.float32),
            pltpu.VMEM((_C * _PITCH,), jnp.float32),
            pltpu.VMEM((_C * D,), jnp.float32),
            pltpu.VMEM((_C * D,), jnp.float32),
            pltpu.VMEM((_C, D), jnp.float32),
            pltpu.VMEM((_C * D,), jnp.float32),
            pltpu.VMEM((_C * D,), jnp.float32),
            pltpu.VMEM((_C, D), jnp.float32),
            pltpu.SemaphoreType.DMA,
            pltpu.SemaphoreType.DMA,
            pltpu.SemaphoreType.DMA,
            pltpu.SemaphoreType.DMA,
            pltpu.SemaphoreType.DMA,
            pltpu.SemaphoreType.DMA,
        ],
    )(h.reshape(B * D), r_id, t.reshape(B * D), rel_weight)


def kernel(h, r_id, t, rel_weight):
    return _complex_score(h, r_id, t, rel_weight)
